# Initial kernel scaffold; baseline (speedup 1.0000x reference)
import jax
import jax.numpy as jnp
from jax.experimental import pallas as pl


def kernel(x, edge_index, edge_attr, Wq, bq, Wk, bk, Wv, bv, We, be, Wg, bg, Wo, bo, edge_w, msg_w, attn_bi, skip_g, ln1g, ln1b, ln1eg, ln1eb, ln2g, ln2b, W1, b1, W2, b2):
    raise NotImplementedError("write your pallas kernel here")


# speedup vs baseline: 13.6269x; 13.6269x over previous
import functools

import jax
import jax.numpy as jnp
from jax import lax
from jax.experimental import pallas as pl
from jax.experimental.pallas import tpu as pltpu
from jax.experimental.pallas import tpu_sc as plsc

N = 10000
E = 320000
DIN = 128
DH = 128
DE = 16
Hh = 4
Dd = 32

NPAD = 10240
DUMMY = N
CW = 128
NTILES = 32
EPAD = 323584
BN = 512
BE = 1024

_INV_SQRT_D = 1.0 / (Dd ** 0.5)


def _ln_rows(xx, g, b):
    mu = jnp.mean(xx, axis=-1, keepdims=True)
    var = jnp.mean((xx - mu) ** 2, axis=-1, keepdims=True)
    return (xx - mu) / jnp.sqrt(var + 1e-5) * g + b


def _node_tables_body(x_ref, ln1g_ref, ln1b_ref, wq_ref, bq_ref, wk_ref,
                      bk_ref, wv_ref, bv_ref, bdk_ref, bdm_ref,
                      q_ref, kv_ref):
    h = _ln_rows(x_ref[...], ln1g_ref[...], ln1b_ref[...])
    q = jnp.dot(h, wq_ref[...], preferred_element_type=jnp.float32) + bq_ref[...]
    k = jnp.dot(h, wk_ref[...], preferred_element_type=jnp.float32) + bk_ref[...]
    v = jnp.dot(h, wv_ref[...], preferred_element_type=jnp.float32) + bv_ref[...]
    q_ref[...] = q
    kv_ref[:, :DH] = jnp.dot(k, bdk_ref[...], preferred_element_type=jnp.float32)
    kv_ref[:, DH:] = jnp.dot(v, bdm_ref[...], preferred_element_type=jnp.float32)


def _node_tables(x_p, ln1g, ln1b, Wq, bq, Wk, bk, Wv, bv, bdk, bdm):
    full = lambda shape: pl.BlockSpec(shape, lambda i: (0,) * len(shape))
    return pl.pallas_call(
        _node_tables_body,
        grid=(NPAD // BN,),
        in_specs=[
            pl.BlockSpec((BN, DIN), lambda i: (i, 0)),
            full((DIN,)), full((DIN,)),
            full((DIN, DH)), full((DH,)),
            full((DIN, DH)), full((DH,)),
            full((DIN, DH)), full((DH,)),
            full((DH, DH)), full((DH, DH)),
        ],
        out_specs=[
            pl.BlockSpec((BN, DH), lambda i: (i, 0)),
            pl.BlockSpec((BN, 2 * DH), lambda i: (i, 0)),
        ],
        out_shape=[
            jax.ShapeDtypeStruct((NPAD, DH), jnp.float32),
            jax.ShapeDtypeStruct((NPAD, 2 * DH), jnp.float32),
        ],
    )(x_p, ln1g, ln1b, Wq, bq, Wk, bk, Wv, bv, bdk, bdm)


def _sc_gather_one(table, idx_flat, width):
    mesh = plsc.VectorSubcoreMesh(core_axis_name="c", subcore_axis_name="s")
    per_tile = EPAD // NTILES
    steps = per_tile // CW

    @functools.partial(
        pl.kernel,
        out_type=jax.ShapeDtypeStruct((EPAD, width), jnp.float32),
        mesh=mesh,
        scratch_types=[
            pltpu.VMEM((CW,), jnp.int32),
            pltpu.VMEM((CW, width), jnp.float32),
            pltpu.SemaphoreType.DMA,
        ],
    )
    def k(tab_hbm, i_hbm, o_hbm, idx_v, rows_v, sem):
        wid = lax.axis_index("s") * 2 + lax.axis_index("c")
        base = wid * per_tile

        @pl.loop(0, steps)
        def _(c):
            off = base + c * CW
            pltpu.sync_copy(i_hbm.at[pl.ds(off, CW)], idx_v)
            pltpu.async_copy(tab_hbm.at[idx_v], rows_v, sem).wait()
            pltpu.sync_copy(rows_v, o_hbm.at[pl.ds(off, CW)])

    return k(table, idx_flat)


def _sc_gather(kvtab, qtab, src_p, dst_p):
    kvsrc = _sc_gather_one(kvtab, src_p, 2 * DH)
    qdst = _sc_gather_one(qtab, dst_p, DH)
    return kvsrc, qdst


def _edge_math_body(kvsrc_ref, qdst_ref, ea_ref, ln1eg_ref, ln1eb_ref,
                    we_ref, be_ref, wg_ref, bg_ref, biasrow_ref,
                    wm_ref, exb_ref):
    lnea = _ln_rows(ea_ref[...], ln1eg_ref[...], ln1eb_ref[...])
    e = jnp.dot(lnea, we_ref[...], preferred_element_type=jnp.float32) + be_ref[...]
    g = jnp.dot(lnea, wg_ref[...], preferred_element_type=jnp.float32) + bg_ref[...]
    kt = kvsrc_ref[:, :DH]
    mv = kvsrc_ref[:, DH:]
    qk = qdst_ref[...] * (kt + e)
    r_i = lax.broadcasted_iota(jnp.int32, (DH, DH), 0)
    c_i = lax.broadcasted_iota(jnp.int32, (DH, DH), 1)
    mh = (r_i // Dd == c_i).astype(jnp.float32)
    mb = (r_i == c_i // Dd).astype(jnp.float32)
    s128 = jnp.dot(qk, mh, preferred_element_type=jnp.float32)
    sc = s128 * _INV_SQRT_D + biasrow_ref[...]
    lane = lax.broadcasted_iota(jnp.int32, sc.shape, 1)
    ex = jnp.exp(sc) * (lane < Hh).astype(jnp.float32)
    exb = jnp.dot(ex, mb, preferred_element_type=jnp.float32)
    msg = mv * jax.nn.sigmoid(g)
    wm_ref[...] = exb * msg
    exb_ref[...] = exb


def _edge_math(kvsrc, qdst, ea_p, ln1eg, ln1eb, We, be, Wg, bg, biasrow):
    full = lambda shape: pl.BlockSpec(shape, lambda i: (0,) * len(shape))
    return pl.pallas_call(
        _edge_math_body,
        grid=(EPAD // BE,),
        in_specs=[
            pl.BlockSpec((BE, 2 * DH), lambda i: (i, 0)),
            pl.BlockSpec((BE, DH), lambda i: (i, 0)),
            pl.BlockSpec((BE, DE), lambda i: (i, 0)),
            full((DE,)), full((DE,)),
            full((DE, DH)), full((DH,)),
            full((DE, DH)), full((DH,)),
            full((1, DH)),
        ],
        out_specs=[pl.BlockSpec((BE, DH), lambda i: (i, 0)),
                   pl.BlockSpec((BE, DH), lambda i: (i, 0))],
        out_shape=[jax.ShapeDtypeStruct((EPAD, DH), jnp.float32),
                   jax.ShapeDtypeStruct((EPAD, DH), jnp.float32)],
    )(kvsrc, qdst, ea_p, ln1eg, ln1eb, We, be, Wg, bg, biasrow)


def _sc_scatter_one(edges, dst_flat, zeros_acc):
    mesh = plsc.VectorSubcoreMesh(core_axis_name="c", subcore_axis_name="s")
    rows_per_tile = NPAD // 16

    per_tile = EPAD // NTILES
    steps = per_tile // CW

    @functools.partial(
        pl.kernel,
        out_type=jax.ShapeDtypeStruct((2, NPAD, DH), jnp.float32),
        mesh=mesh,
        scratch_types=[
            pltpu.VMEM((CW,), jnp.int32),
            pltpu.VMEM((CW, DH), jnp.float32),
            pltpu.VMEM_SHARED((NPAD, DH), jnp.float32),
        ],
    )
    def k(edge_hbm, dst_hbm, zeros_hbm, out_hbm, idx_v, rows_v, acc):
        cid = lax.axis_index("c")
        sid = lax.axis_index("s")
        wid = sid * 2 + cid
        base_e = wid * per_tile
        base_n = sid * rows_per_tile
        pltpu.sync_copy(zeros_hbm.at[pl.ds(base_n, rows_per_tile)],
                        acc.at[pl.ds(base_n, rows_per_tile)])
        plsc.subcore_barrier()

        @pl.loop(0, steps)
        def _(c):
            off = base_e + c * CW
            pltpu.sync_copy(dst_hbm.at[pl.ds(off, CW)], idx_v)
            pltpu.sync_copy(edge_hbm.at[pl.ds(off, CW)], rows_v)
            pltpu.sync_copy(rows_v, acc.at[idx_v], add=True)

        plsc.subcore_barrier()
        pltpu.sync_copy(acc.at[pl.ds(base_n, rows_per_tile)],
                        out_hbm.at[cid, pl.ds(base_n, rows_per_tile)])

    return k(edges, dst_flat, zeros_acc)


def _sc_scatter(wm, exb, dst_flat, zeros_acc):
    pw = _sc_scatter_one(wm, dst_flat, zeros_acc)
    pe = _sc_scatter_one(exb, dst_flat, zeros_acc)
    return pw, pe


def _final_body(pw_ref, pe_ref, x_ref, skipg_ref, wo_ref, bo_ref, ln2g_ref,
                ln2b_ref, w1_ref, b1_ref, w2_ref, b2_ref, out_ref):
    aggw = pw_ref[0] + pw_ref[1]
    denb = pe_ref[0] + pe_ref[1]
    agg = aggw / (denb + 1e-16)
    h_attn = jnp.dot(agg, wo_ref[...], preferred_element_type=jnp.float32) + bo_ref[...]
    beta = jax.nn.sigmoid(skipg_ref[...])
    h2 = beta * h_attn + (1.0 - beta) * x_ref[...]
    hn = _ln_rows(h2, ln2g_ref[...], ln2b_ref[...])
    ff = jnp.dot(jax.nn.gelu(
        jnp.dot(hn, w1_ref[...], preferred_element_type=jnp.float32) + b1_ref[...]),
        w2_ref[...], preferred_element_type=jnp.float32) + b2_ref[...]
    out_ref[...] = h2 + ff


def _final(pw, pe, x_p, skipg_row, Wo, bo, ln2g, ln2b, W1, b1, W2, b2):
    full = lambda shape: pl.BlockSpec(shape, lambda i: (0,) * len(shape))
    return pl.pallas_call(
        _final_body,
        grid=(NPAD // BN,),
        in_specs=[
            pl.BlockSpec((2, BN, DH), lambda i: (0, i, 0)),
            pl.BlockSpec((2, BN, DH), lambda i: (0, i, 0)),
            pl.BlockSpec((BN, DIN), lambda i: (i, 0)),
            full((1, DH)),
            full((DH, DH)), full((DH,)),
            full((DH,)), full((DH,)),
            full((DH, 2 * DH)), full((2 * DH,)),
            full((2 * DH, DH)), full((DH,)),
        ],
        out_specs=[pl.BlockSpec((BN, DH), lambda i: (i, 0))],
        out_shape=[jax.ShapeDtypeStruct((NPAD, DH), jnp.float32)],
    )(pw, pe, x_p, skipg_row, Wo, bo, ln2g, ln2b, W1, b1, W2, b2)[0]


def kernel(x, edge_index, edge_attr, Wq, bq, Wk, bk, Wv, bv, We, be, Wg, bg,
           Wo, bo, edge_w, msg_w, attn_bi, skip_g, ln1g, ln1b, ln1eg, ln1eb,
           ln2g, ln2b, W1, b1, W2, b2):
    src = edge_index[0]
    dst = edge_index[1]

    x_p = jnp.pad(x, ((0, NPAD - N), (0, 0)))
    src_p = jnp.concatenate([src, jnp.zeros((EPAD - E,), jnp.int32)])
    dst_p = jnp.concatenate([dst, jnp.full((EPAD - E,), DUMMY, jnp.int32)])
    ea_p = jnp.pad(edge_attr, ((0, EPAD - E), (0, 0)))
    bdk = jnp.zeros((DH, DH), jnp.float32)
    bdm = jnp.zeros((DH, DH), jnp.float32)
    for hh in range(Hh):
        sl = slice(hh * Dd, (hh + 1) * Dd)
        bdk = bdk.at[sl, sl].set(edge_w[0, hh])
        bdm = bdm.at[sl, sl].set(msg_w[0, hh])
    biasrow = jnp.zeros((1, DH), jnp.float32).at[0, :Hh].set(attn_bi[:, 0])
    skipg_row = jnp.broadcast_to(skip_g, (1, DH)).astype(jnp.float32)
    zeros_acc = jnp.zeros((NPAD, DH), jnp.float32)

    qtab, kvtab = _node_tables(x_p, ln1g, ln1b, Wq, bq, Wk, bk, Wv, bv,
                               bdk, bdm)
    kvsrc, qdst = _sc_gather(kvtab, qtab, src_p, dst_p)
    wm, exb = _edge_math(kvsrc, qdst, ea_p, ln1eg, ln1eb, We, be, Wg, bg,
                         biasrow)
    pw, pe = _sc_scatter(wm, exb, dst_p, zeros_acc)
    out = _final(pw, pe, x_p, skipg_row, Wo, bo, ln2g, ln2b, W1, b1, W2, b2)
    return out[:N]
